# baseline (speedup 1.0000x reference)
import functools

import jax
import jax.numpy as jnp
from jax.experimental import pallas as pl
from jax.experimental.pallas import tpu as pltpu

_BN_INV = 0.99999500003749969


def _encoder_body(rect_hbm, recog_hbm, oq_ref,
                  wrect_ref, brect_ref, wrecog_ref, brecog_ref,
                  w1_ref, b1_ref, g1_ref, bt1_ref,
                  w2t_ref, b2_ref, g2_ref, bt2_ref,
                  out_ref, rbuf, qbuf, sems, *, nt, sp, t, sem_w):
    i = pl.program_id(0)
    ng = pl.num_programs(0)

    def copies(slot, base):
        cs = []
        for j in range(sp):
            w, h = j // 2, j % 2
            cs.append(pltpu.make_async_copy(
                rect_hbm.at[pl.ds(base, nt), w, h],
                rbuf.at[slot, j], sems.at[slot, j]))
        for k in range(t):
            cs.append(pltpu.make_async_copy(
                recog_hbm.at[pl.ds(base, nt), k],
                qbuf.at[slot, k], sems.at[slot, sp + k]))
        return cs

    @pl.when(i == 0)
    def _prologue():
        for c in copies(0, 0):
            c.start()

    @pl.when(i + 1 < ng)
    def _prefetch():
        for c in copies((i + 1) % 2, (i + 1) * nt):
            c.start()

    slot = i % 2
    for c in copies(slot, i * nt):
        c.wait()

    rsum = rbuf[slot, 0]
    for j in range(1, sp):
        rsum = rsum + rbuf[slot, j]
    avg = jnp.dot(rsum * (1.0 / sp), wrect_ref[...],
                  preferred_element_type=jnp.float32) + brect_ref[...]
    qsum = qbuf[slot, 0]
    for k in range(1, t):
        qsum = qsum + qbuf[slot, k]
    rec = jnp.dot(qsum * (1.0 / t), wrecog_ref[...],
                  preferred_element_type=jnp.float32) + brecog_ref[...]
    x = jnp.concatenate([avg, rec], axis=1)
    s1 = g1_ref[...] * _BN_INV
    c1 = b1_ref[...] * s1 + bt1_ref[...]
    h = jnp.maximum(jnp.dot(x, w1_ref[...],
                            preferred_element_type=jnp.float32) * s1 + c1, 0.0)
    s2 = g2_ref[...] * _BN_INV
    c2 = b2_ref[...] * s2 + bt2_ref[...]
    x2 = jax.lax.dot_general(h, w2t_ref[...], (((1,), (1,)), ((), ())),
                             preferred_element_type=jnp.float32)
    sem = jnp.maximum(x2 * s2 + c2, 0.0)

    oq = oq_ref[...] * (1.0 / 1024.0)
    c = [oq[:, k:k + 1] for k in range(8)]
    d1x = (c[2] + c[4] - c[0] - c[6]) * 0.5
    d1y = (c[3] + c[5] - c[1] - c[7]) * 0.5
    wd = jnp.sqrt(d1x * d1x + d1y * d1y)
    den = jnp.maximum(wd, 1e-6)
    d1xn = d1x / den
    d1yn = d1y / den
    hx = (c[6] - c[0] + c[4] - c[2]) * 0.5
    hy = (c[7] - c[1] + c[5] - c[3]) * 0.5
    hts = jnp.sqrt(hx * hx + hy * hy)
    geom = jnp.concatenate([oq, d1xn, d1yn, -d1yn, d1xn, wd, hts], axis=1)

    out_ref[...] = jnp.concatenate([sem[:, :sem_w], geom], axis=1)


def kernel(rectified_quads, original_quads, region_counts, recog_features,
           W_rect, b_rect, W_recog, b_recog,
           W1, b1, g1, bt1, W2, b2, g2, bt2):
    del region_counts
    rectified_quads = rectified_quads.astype(jnp.float32)
    recog_features = recog_features.astype(jnp.float32)
    n = rectified_quads.shape[0]
    hh, ww = rectified_quads.shape[2], rectified_quads.shape[3]
    sp = hh * ww
    t = recog_features.shape[1]
    d = recog_features.shape[2]
    ch = rectified_quads.shape[1]
    sem_w = W2.shape[1]
    out_w = sem_w + 14

    rect_t = jnp.transpose(rectified_quads, (0, 3, 2, 1))
    w2t = jnp.transpose(W2, (1, 0))
    oq8 = original_quads.reshape(n, 8)

    nt = 200
    assert n % nt == 0, (n, nt)
    grid = (n // nt,)

    body = functools.partial(_encoder_body, nt=nt, sp=sp, t=t, sem_w=sem_w)
    rep = lambda i: (0, 0)
    hbm = pl.BlockSpec(memory_space=pltpu.MemorySpace.HBM)
    row = lambda v: v.reshape(1, -1)
    out = pl.pallas_call(
        body,
        grid=grid,
        in_specs=[
            hbm, hbm,
            pl.BlockSpec((nt, 8), lambda i: (i, 0)),
            pl.BlockSpec(W_rect.shape, rep),
            pl.BlockSpec((1, ch), rep),
            pl.BlockSpec(W_recog.shape, rep),
            pl.BlockSpec((1, ch), rep),
            pl.BlockSpec(W1.shape, rep),
            pl.BlockSpec((1, 2 * ch), rep),
            pl.BlockSpec((1, 2 * ch), rep),
            pl.BlockSpec((1, 2 * ch), rep),
            pl.BlockSpec(w2t.shape, rep),
            pl.BlockSpec((1, sem_w), rep),
            pl.BlockSpec((1, sem_w), rep),
            pl.BlockSpec((1, sem_w), rep),
        ],
        out_specs=pl.BlockSpec((nt, out_w), lambda i: (i, 0)),
        out_shape=jax.ShapeDtypeStruct((n, out_w), jnp.float32),
        scratch_shapes=[
            pltpu.VMEM((2, sp, nt, ch), jnp.float32),
            pltpu.VMEM((2, t, nt, d), jnp.float32),
            pltpu.SemaphoreType.DMA((2, sp + t)),
        ],
        compiler_params=pltpu.CompilerParams(
            dimension_semantics=("arbitrary",)),
    )(rect_t, recog_features, oq8,
      W_rect, row(b_rect), W_recog, row(b_recog),
      W1, row(b1), row(g1), row(bt1),
      w2t, row(b2), row(g2), row(bt2))
    return out


# speedup vs baseline: 1.1775x; 1.1775x over previous
import functools

import jax
import jax.numpy as jnp
from jax.experimental import pallas as pl
from jax.experimental.pallas import tpu as pltpu

_BN_INV = 0.99999500003749969


def _encoder_body(rect_hbm, recog_hbm, oq_ref,
                  wrect_ref, brect_ref, wrecog_ref, brecog_ref,
                  w1_ref, b1_ref, g1_ref, bt1_ref,
                  w2t_ref, b2_ref, g2_ref, bt2_ref,
                  out_ref, rbuf, qbuf, sems, *, nt, sp, t, sem_w):
    i = pl.program_id(0)
    ng = pl.num_programs(0)

    def copies(slot, base):
        cs = []
        for j in range(sp):
            w, h = j // 2, j % 2
            cs.append(pltpu.make_async_copy(
                rect_hbm.at[pl.ds(base, nt), w, h],
                rbuf.at[slot, j], sems.at[slot, j]))
        for k in range(t):
            cs.append(pltpu.make_async_copy(
                recog_hbm.at[pl.ds(base, nt), k],
                qbuf.at[slot, k], sems.at[slot, sp + k]))
        return cs

    @pl.when(i == 0)
    def _prologue():
        for c in copies(0, 0):
            c.start()

    @pl.when(i + 1 < ng)
    def _prefetch():
        for c in copies((i + 1) % 2, (i + 1) * nt):
            c.start()

    slot = i % 2
    for c in copies(slot, i * nt):
        c.wait()

    rsum = rbuf[slot, 0]
    for j in range(1, sp):
        rsum = rsum + rbuf[slot, j]
    avg = jnp.dot(rsum * (1.0 / sp), wrect_ref[...],
                  preferred_element_type=jnp.float32) + brect_ref[...]
    qsum = qbuf[slot, 0]
    for k in range(1, t):
        qsum = qsum + qbuf[slot, k]
    rec = jnp.dot(qsum * (1.0 / t), wrecog_ref[...],
                  preferred_element_type=jnp.float32) + brecog_ref[...]
    x = jnp.concatenate([avg, rec], axis=1)
    s1 = g1_ref[...] * _BN_INV
    c1 = b1_ref[...] * s1 + bt1_ref[...]
    h = jnp.maximum(jnp.dot(x, w1_ref[...],
                            preferred_element_type=jnp.float32) * s1 + c1, 0.0)
    s2 = g2_ref[...] * _BN_INV
    c2 = b2_ref[...] * s2 + bt2_ref[...]
    x2 = jax.lax.dot_general(h, w2t_ref[...], (((1,), (1,)), ((), ())),
                             preferred_element_type=jnp.float32)
    sem = jnp.maximum(x2 * s2 + c2, 0.0)

    oq = oq_ref[...] * (1.0 / 1024.0)
    c = [oq[:, k:k + 1] for k in range(8)]
    d1x = (c[2] + c[4] - c[0] - c[6]) * 0.5
    d1y = (c[3] + c[5] - c[1] - c[7]) * 0.5
    wd = jnp.sqrt(d1x * d1x + d1y * d1y)
    den = jnp.maximum(wd, 1e-6)
    d1xn = d1x / den
    d1yn = d1y / den
    hx = (c[6] - c[0] + c[4] - c[2]) * 0.5
    hy = (c[7] - c[1] + c[5] - c[3]) * 0.5
    hts = jnp.sqrt(hx * hx + hy * hy)
    geom = jnp.concatenate([oq, d1xn, d1yn, -d1yn, d1xn, wd, hts], axis=1)

    out_ref[...] = jnp.concatenate([sem[:, :sem_w], geom], axis=1)


def kernel(rectified_quads, original_quads, region_counts, recog_features,
           W_rect, b_rect, W_recog, b_recog,
           W1, b1, g1, bt1, W2, b2, g2, bt2):
    del region_counts
    rectified_quads = rectified_quads.astype(jnp.float32)
    recog_features = recog_features.astype(jnp.float32)
    n = rectified_quads.shape[0]
    hh, ww = rectified_quads.shape[2], rectified_quads.shape[3]
    sp = hh * ww
    t = recog_features.shape[1]
    d = recog_features.shape[2]
    ch = rectified_quads.shape[1]
    sem_w = W2.shape[1]
    out_w = sem_w + 14

    rect_t = jnp.transpose(rectified_quads, (0, 3, 2, 1))
    w2t = jnp.transpose(W2, (1, 0))
    oq8 = original_quads.reshape(n, 8)

    nt = 1000
    assert n % nt == 0, (n, nt)
    grid = (n // nt,)

    body = functools.partial(_encoder_body, nt=nt, sp=sp, t=t, sem_w=sem_w)
    rep = lambda i: (0, 0)
    hbm = pl.BlockSpec(memory_space=pltpu.MemorySpace.HBM)
    row = lambda v: v.reshape(1, -1)
    out = pl.pallas_call(
        body,
        grid=grid,
        in_specs=[
            hbm, hbm,
            pl.BlockSpec((nt, 8), lambda i: (i, 0)),
            pl.BlockSpec(W_rect.shape, rep),
            pl.BlockSpec((1, ch), rep),
            pl.BlockSpec(W_recog.shape, rep),
            pl.BlockSpec((1, ch), rep),
            pl.BlockSpec(W1.shape, rep),
            pl.BlockSpec((1, 2 * ch), rep),
            pl.BlockSpec((1, 2 * ch), rep),
            pl.BlockSpec((1, 2 * ch), rep),
            pl.BlockSpec(w2t.shape, rep),
            pl.BlockSpec((1, sem_w), rep),
            pl.BlockSpec((1, sem_w), rep),
            pl.BlockSpec((1, sem_w), rep),
        ],
        out_specs=pl.BlockSpec((nt, out_w), lambda i: (i, 0)),
        out_shape=jax.ShapeDtypeStruct((n, out_w), jnp.float32),
        scratch_shapes=[
            pltpu.VMEM((2, sp, nt, ch), jnp.float32),
            pltpu.VMEM((2, t, nt, d), jnp.float32),
            pltpu.SemaphoreType.DMA((2, sp + t)),
        ],
        compiler_params=pltpu.CompilerParams(
            dimension_semantics=("arbitrary",)),
    )(rect_t, recog_features, oq8,
      W_rect, row(b_rect), W_recog, row(b_recog),
      W1, row(b1), row(g1), row(bt1),
      w2t, row(b2), row(g2), row(bt2))
    return out


# speedup vs baseline: 1.1999x; 1.0190x over previous
import functools

import jax
import jax.numpy as jnp
from jax.experimental import pallas as pl
from jax.experimental.pallas import tpu as pltpu

_BN_INV = 0.99999500003749969


def _encoder_body(rect_hbm, recog_hbm, oq_ref,
                  wrect_ref, brect_ref, wrecog_ref, brecog_ref,
                  w1_ref, b1_ref, g1_ref, bt1_ref,
                  w2t_ref, b2_ref, g2_ref, bt2_ref,
                  out_ref, rbuf, qbuf, sems, *, nt, sp, t, sem_w):
    i = pl.program_id(0)
    ng = pl.num_programs(0)

    def copies(slot, base):
        cs = []
        for j in range(sp):
            w, h = j // 2, j % 2
            cs.append(pltpu.make_async_copy(
                rect_hbm.at[pl.ds(base, nt), w, h],
                rbuf.at[slot, j], sems.at[slot, j]))
        for k in range(t):
            cs.append(pltpu.make_async_copy(
                recog_hbm.at[pl.ds(base, nt), k],
                qbuf.at[slot, k], sems.at[slot, sp + k]))
        return cs

    @pl.when(i == 0)
    def _prologue():
        for c in copies(0, 0):
            c.start()
        for c in copies(1, nt):
            c.start()

    @pl.when(i + 2 < ng)
    def _prefetch():
        for c in copies((i + 2) % 3, (i + 2) * nt):
            c.start()

    slot = i % 3
    for c in copies(slot, i * nt):
        c.wait()

    rsum = rbuf[slot, 0]
    for j in range(1, sp):
        rsum = rsum + rbuf[slot, j]
    avg = jnp.dot(rsum * (1.0 / sp), wrect_ref[...],
                  preferred_element_type=jnp.float32) + brect_ref[...]
    qsum = qbuf[slot, 0]
    for k in range(1, t):
        qsum = qsum + qbuf[slot, k]
    rec = jnp.dot(qsum * (1.0 / t), wrecog_ref[...],
                  preferred_element_type=jnp.float32) + brecog_ref[...]
    x = jnp.concatenate([avg, rec], axis=1)
    s1 = g1_ref[...] * _BN_INV
    c1 = b1_ref[...] * s1 + bt1_ref[...]
    h = jnp.maximum(jnp.dot(x, w1_ref[...],
                            preferred_element_type=jnp.float32) * s1 + c1, 0.0)
    s2 = g2_ref[...] * _BN_INV
    c2 = b2_ref[...] * s2 + bt2_ref[...]
    x2 = jax.lax.dot_general(h, w2t_ref[...], (((1,), (1,)), ((), ())),
                             preferred_element_type=jnp.float32)
    sem = jnp.maximum(x2 * s2 + c2, 0.0)

    oq = oq_ref[...] * (1.0 / 1024.0)
    c = [oq[:, k:k + 1] for k in range(8)]
    d1x = (c[2] + c[4] - c[0] - c[6]) * 0.5
    d1y = (c[3] + c[5] - c[1] - c[7]) * 0.5
    wd = jnp.sqrt(d1x * d1x + d1y * d1y)
    den = jnp.maximum(wd, 1e-6)
    d1xn = d1x / den
    d1yn = d1y / den
    hx = (c[6] - c[0] + c[4] - c[2]) * 0.5
    hy = (c[7] - c[1] + c[5] - c[3]) * 0.5
    hts = jnp.sqrt(hx * hx + hy * hy)
    geom = jnp.concatenate([oq, d1xn, d1yn, -d1yn, d1xn, wd, hts], axis=1)

    out_ref[...] = jnp.concatenate([sem[:, :sem_w], geom], axis=1)


def kernel(rectified_quads, original_quads, region_counts, recog_features,
           W_rect, b_rect, W_recog, b_recog,
           W1, b1, g1, bt1, W2, b2, g2, bt2):
    del region_counts
    rectified_quads = rectified_quads.astype(jnp.float32)
    recog_features = recog_features.astype(jnp.float32)
    n = rectified_quads.shape[0]
    hh, ww = rectified_quads.shape[2], rectified_quads.shape[3]
    sp = hh * ww
    t = recog_features.shape[1]
    d = recog_features.shape[2]
    ch = rectified_quads.shape[1]
    sem_w = W2.shape[1]
    out_w = sem_w + 14

    rect_t = jnp.transpose(rectified_quads, (0, 3, 2, 1))
    w2t = jnp.transpose(W2, (1, 0))
    oq8 = original_quads.reshape(n, 8)

    nt = 400
    assert n % nt == 0, (n, nt)
    grid = (n // nt,)

    body = functools.partial(_encoder_body, nt=nt, sp=sp, t=t, sem_w=sem_w)
    rep = lambda i: (0, 0)
    hbm = pl.BlockSpec(memory_space=pltpu.MemorySpace.HBM)
    row = lambda v: v.reshape(1, -1)
    out = pl.pallas_call(
        body,
        grid=grid,
        in_specs=[
            hbm, hbm,
            pl.BlockSpec((nt, 8), lambda i: (i, 0)),
            pl.BlockSpec(W_rect.shape, rep),
            pl.BlockSpec((1, ch), rep),
            pl.BlockSpec(W_recog.shape, rep),
            pl.BlockSpec((1, ch), rep),
            pl.BlockSpec(W1.shape, rep),
            pl.BlockSpec((1, 2 * ch), rep),
            pl.BlockSpec((1, 2 * ch), rep),
            pl.BlockSpec((1, 2 * ch), rep),
            pl.BlockSpec(w2t.shape, rep),
            pl.BlockSpec((1, sem_w), rep),
            pl.BlockSpec((1, sem_w), rep),
            pl.BlockSpec((1, sem_w), rep),
        ],
        out_specs=pl.BlockSpec((nt, out_w), lambda i: (i, 0)),
        out_shape=jax.ShapeDtypeStruct((n, out_w), jnp.float32),
        scratch_shapes=[
            pltpu.VMEM((3, sp, nt, ch), jnp.float32),
            pltpu.VMEM((3, t, nt, d), jnp.float32),
            pltpu.SemaphoreType.DMA((3, sp + t)),
        ],
        compiler_params=pltpu.CompilerParams(
            dimension_semantics=("arbitrary",)),
    )(rect_t, recog_features, oq8,
      W_rect, row(b_rect), W_recog, row(b_recog),
      W1, row(b1), row(g1), row(bt1),
      w2t, row(b2), row(g2), row(bt2))
    return out


# speedup vs baseline: 1.2086x; 1.0073x over previous
import functools

import jax
import jax.numpy as jnp
from jax.experimental import pallas as pl
from jax.experimental.pallas import tpu as pltpu

_BN_INV = 0.99999500003749969


def _encoder_body(rect_hbm, recog_hbm, oq_ref,
                  wrect_ref, brect_ref, wrecog_ref, brecog_ref,
                  w1_ref, b1_ref, g1_ref, bt1_ref,
                  w2t_ref, b2_ref, g2_ref, bt2_ref,
                  out_ref, rbuf, qbuf, sems, *, nt, sp, t, sem_w):
    i = pl.program_id(0)
    ng = pl.num_programs(0)

    def copies(slot, base):
        cs = []
        for k in range(t):
            cs.append(pltpu.make_async_copy(
                recog_hbm.at[pl.ds(base, nt), k],
                qbuf.at[slot, k], sems.at[slot, sp + k]))
        for j in range(sp):
            w, h = j // 2, j % 2
            cs.append(pltpu.make_async_copy(
                rect_hbm.at[pl.ds(base, nt), w, h],
                rbuf.at[slot, j], sems.at[slot, j]))
        return cs

    @pl.when(i == 0)
    def _prologue():
        for c in copies(0, 0):
            c.start()

    @pl.when(i + 1 < ng)
    def _prefetch():
        for c in copies((i + 1) % 2, (i + 1) * nt):
            c.start()

    slot = i % 2
    for c in copies(slot, i * nt):
        c.wait()

    rsum = rbuf[slot, 0]
    for j in range(1, sp):
        rsum = rsum + rbuf[slot, j]
    avg = jnp.dot(rsum * (1.0 / sp), wrect_ref[...],
                  preferred_element_type=jnp.float32) + brect_ref[...]
    qsum = qbuf[slot, 0]
    for k in range(1, t):
        qsum = qsum + qbuf[slot, k]
    rec = jnp.dot(qsum * (1.0 / t), wrecog_ref[...],
                  preferred_element_type=jnp.float32) + brecog_ref[...]
    x = jnp.concatenate([avg, rec], axis=1)
    s1 = g1_ref[...] * _BN_INV
    c1 = b1_ref[...] * s1 + bt1_ref[...]
    h = jnp.maximum(jnp.dot(x, w1_ref[...],
                            preferred_element_type=jnp.float32) * s1 + c1, 0.0)
    s2 = g2_ref[...] * _BN_INV
    c2 = b2_ref[...] * s2 + bt2_ref[...]
    x2 = jax.lax.dot_general(h, w2t_ref[...], (((1,), (1,)), ((), ())),
                             preferred_element_type=jnp.float32)
    sem = jnp.maximum(x2 * s2 + c2, 0.0)

    oq = oq_ref[...] * (1.0 / 1024.0)
    c = [oq[:, k:k + 1] for k in range(8)]
    d1x = (c[2] + c[4] - c[0] - c[6]) * 0.5
    d1y = (c[3] + c[5] - c[1] - c[7]) * 0.5
    wd = jnp.sqrt(d1x * d1x + d1y * d1y)
    den = jnp.maximum(wd, 1e-6)
    d1xn = d1x / den
    d1yn = d1y / den
    hx = (c[6] - c[0] + c[4] - c[2]) * 0.5
    hy = (c[7] - c[1] + c[5] - c[3]) * 0.5
    hts = jnp.sqrt(hx * hx + hy * hy)
    geom = jnp.concatenate([oq, d1xn, d1yn, -d1yn, d1xn, wd, hts], axis=1)

    out_ref[...] = jnp.concatenate([sem[:, :sem_w], geom], axis=1)


def kernel(rectified_quads, original_quads, region_counts, recog_features,
           W_rect, b_rect, W_recog, b_recog,
           W1, b1, g1, bt1, W2, b2, g2, bt2):
    del region_counts
    rectified_quads = rectified_quads.astype(jnp.float32)
    recog_features = recog_features.astype(jnp.float32)
    n = rectified_quads.shape[0]
    hh, ww = rectified_quads.shape[2], rectified_quads.shape[3]
    sp = hh * ww
    t = recog_features.shape[1]
    d = recog_features.shape[2]
    ch = rectified_quads.shape[1]
    sem_w = W2.shape[1]
    out_w = sem_w + 14

    rect_t = jnp.transpose(rectified_quads, (0, 3, 2, 1))
    w2t = jnp.transpose(W2, (1, 0))
    oq8 = original_quads.reshape(n, 8)

    nt = 400
    assert n % nt == 0, (n, nt)
    grid = (n // nt,)

    body = functools.partial(_encoder_body, nt=nt, sp=sp, t=t, sem_w=sem_w)
    rep = lambda i: (0, 0)
    hbm = pl.BlockSpec(memory_space=pltpu.MemorySpace.HBM)
    row = lambda v: v.reshape(1, -1)
    out = pl.pallas_call(
        body,
        grid=grid,
        in_specs=[
            hbm, hbm,
            pl.BlockSpec((nt, 8), lambda i: (i, 0)),
            pl.BlockSpec(W_rect.shape, rep),
            pl.BlockSpec((1, ch), rep),
            pl.BlockSpec(W_recog.shape, rep),
            pl.BlockSpec((1, ch), rep),
            pl.BlockSpec(W1.shape, rep),
            pl.BlockSpec((1, 2 * ch), rep),
            pl.BlockSpec((1, 2 * ch), rep),
            pl.BlockSpec((1, 2 * ch), rep),
            pl.BlockSpec(w2t.shape, rep),
            pl.BlockSpec((1, sem_w), rep),
            pl.BlockSpec((1, sem_w), rep),
            pl.BlockSpec((1, sem_w), rep),
        ],
        out_specs=pl.BlockSpec((nt, out_w), lambda i: (i, 0)),
        out_shape=jax.ShapeDtypeStruct((n, out_w), jnp.float32),
        scratch_shapes=[
            pltpu.VMEM((2, sp, nt, ch), jnp.float32),
            pltpu.VMEM((2, t, nt, d), jnp.float32),
            pltpu.SemaphoreType.DMA((2, sp + t)),
        ],
        compiler_params=pltpu.CompilerParams(
            dimension_semantics=("arbitrary",)),
    )(rect_t, recog_features, oq8,
      W_rect, row(b_rect), W_recog, row(b_recog),
      W1, row(b1), row(g1), row(bt1),
      w2t, row(b2), row(g2), row(bt2))
    return out
